# baseline (speedup 1.0000x reference)
import jax
import jax.numpy as jnp
from jax.experimental import pallas as pl

N = 1024
D = 128
BM = 1024


def _gin_kernel(a_ref, x_ref, w_ref, b_ref, o_ref):
    i = pl.program_id(0)
    a = a_ref[...].astype(jnp.bfloat16)
    x = x_ref[...].astype(jnp.bfloat16)
    aggr = jnp.dot(a, x, preferred_element_type=jnp.float32)
    h = aggr + x_ref[pl.ds(i * BM, BM), :]
    out = jax.lax.dot_general(
        h.astype(jnp.bfloat16), w_ref[...].astype(jnp.bfloat16),
        (((1,), (1,)), ((), ())), preferred_element_type=jnp.float32)
    o_ref[...] = jnp.maximum(out + b_ref[...], 0.0)


def kernel(A, X, W, b):
    return pl.pallas_call(
        _gin_kernel,
        grid=(N // BM,),
        in_specs=[
            pl.BlockSpec((BM, N), lambda i: (i, 0)),
            pl.BlockSpec((N, D), lambda i: (0, 0)),
            pl.BlockSpec((D, D), lambda i: (0, 0)),
            pl.BlockSpec((1, D), lambda i: (0, 0)),
        ],
        out_specs=pl.BlockSpec((BM, D), lambda i: (i, 0)),
        out_shape=jax.ShapeDtypeStruct((N, D), jnp.float32),
    )(A, X, W, b.reshape(1, D))


# speedup vs baseline: 1.0934x; 1.0934x over previous
import jax
import jax.numpy as jnp
from jax.experimental import pallas as pl

N = 1024
D = 128
BM = 512


def _gin_kernel(a_ref, x_ref, w_ref, b_ref, o_ref):
    i = pl.program_id(0)
    aggr = jnp.dot(a_ref[...], x_ref[...], preferred_element_type=jnp.float32)
    h = aggr + x_ref[pl.ds(i * BM, BM), :]
    out = jax.lax.dot_general(
        h, w_ref[...], (((1,), (1,)), ((), ())),
        preferred_element_type=jnp.float32)
    o_ref[...] = jnp.maximum(out + b_ref[...], 0.0)


def kernel(A, X, W, b):
    return pl.pallas_call(
        _gin_kernel,
        grid=(N // BM,),
        in_specs=[
            pl.BlockSpec((BM, N), lambda i: (i, 0)),
            pl.BlockSpec((N, D), lambda i: (0, 0)),
            pl.BlockSpec((D, D), lambda i: (0, 0)),
            pl.BlockSpec((1, D), lambda i: (0, 0)),
        ],
        out_specs=pl.BlockSpec((BM, D), lambda i: (i, 0)),
        out_shape=jax.ShapeDtypeStruct((N, D), jnp.float32),
    )(A, X, W, b.reshape(1, D))
